# baseline (speedup 1.0000x reference)
import functools

import jax
import jax.numpy as jnp
from jax import lax
from jax.experimental import pallas as pl
from jax.experimental.pallas import tpu as pltpu
from jax.experimental.pallas import tpu_sc as plsc

_N = 10000
_E = 320000
_D = 128
_EPS = 1e-5
_BR = 1000


_K = 128
_NITER = 80


def _make_seg_sum():
    nc, ns = 2, 16
    k = _K
    niter = _NITER
    ch = 80
    ncht = _N // ch
    tpt = -(-ncht // ns)
    nacc = _N + 8
    mesh = plsc.VectorSubcoreMesh(
        core_axis_name="c", subcore_axis_name="s",
        num_cores=nc, num_subcores=ns)

    @functools.partial(
        pl.kernel,
        mesh=mesh,
        out_type=jax.ShapeDtypeStruct((2 * _N, _D), jnp.float32),
        scratch_types=[
            pltpu.MemorySpace.VMEM_SHARED((nacc, _D), jnp.float32),
            pltpu.MemorySpace.VMEM((niter, k), jnp.int32),
            pltpu.MemorySpace.VMEM((niter, k), jnp.int32),
            pltpu.MemorySpace.VMEM((k, _D), jnp.float32),
            pltpu.SemaphoreType.DMA,
            pltpu.SemaphoreType.DMA,
        ],
    )
    def seg_sum(h_hbm, src_hbm, dst_hbm, out_hbm, acc, sidx, didx, rows,
                gsem, isem):
        cid = lax.axis_index("c")
        sid = lax.axis_index("s")
        wid = sid * nc + cid

        cps = pltpu.async_copy(src_hbm.at[wid], sidx, isem)
        cpd = pltpu.async_copy(dst_hbm.at[wid], didx, isem)

        def init_body(t, _):
            cix = sid + t * ns

            @pl.when(cix < ncht)
            def _():
                r0 = cix * ch
                pltpu.sync_copy(h_hbm.at[pl.ds(r0, ch)],
                                rows.at[pl.ds(0, ch)])
                pltpu.sync_copy(rows.at[pl.ds(0, ch)],
                                acc.at[pl.ds(r0, ch)])
            return 0

        lax.fori_loop(0, tpt, init_body, 0)
        cps.wait()
        cpd.wait()
        plsc.subcore_barrier()

        def edge_body(j, _):
            pltpu.async_copy(h_hbm.at[sidx.at[j]], rows, gsem).wait()
            pltpu.sync_copy(rows, acc.at[didx.at[j]], add=True)
            return 0

        lax.fori_loop(0, niter, edge_body, 0)
        plsc.subcore_barrier()

        def wb_body(t, _):
            cix = sid + t * ns

            @pl.when(cix < ncht)
            def _():
                r0 = cix * ch
                pltpu.sync_copy(acc.at[pl.ds(r0, ch)],
                                rows.at[pl.ds(0, ch)])
                pltpu.sync_copy(rows.at[pl.ds(0, ch)],
                                out_hbm.at[pl.ds(cid * _N + r0, ch)])
            return 0

        lax.fori_loop(0, tpt, wb_body, 0)

    return seg_sum


def _mmA_body(acc_ref, h_ref, w_ref, b_ref, t_ref, st_ref):
    i = pl.program_id(0)
    pooled = acc_ref[0] + acc_ref[1] - h_ref[...]
    t = jnp.dot(pooled, w_ref[...], preferred_element_type=jnp.float32)
    t = t + b_ref[0:1, :]
    t_ref[...] = t
    s1 = jnp.sum(t, axis=0, keepdims=True)
    s2 = jnp.sum(t * t, axis=0, keepdims=True)
    s3 = jnp.sum(h_ref[...], axis=0, keepdims=True)
    st = jnp.concatenate(
        [s1, s2, s3, jnp.zeros((5, _D), jnp.float32)], axis=0)

    @pl.when(i == 0)
    def _():
        st_ref[...] = st

    @pl.when(i != 0)
    def _():
        st_ref[...] = st_ref[...] + st


def _mmA(acc, h, w1, b1):
    return pl.pallas_call(
        _mmA_body,
        grid=(_N // _BR,),
        in_specs=[
            pl.BlockSpec((2, _BR, _D), lambda i: (0, i, 0)),
            pl.BlockSpec((_BR, _D), lambda i: (i, 0)),
            pl.BlockSpec((_D, _D), lambda i: (0, 0)),
            pl.BlockSpec((1, _D), lambda i: (0, 0)),
        ],
        out_specs=[
            pl.BlockSpec((_BR, _D), lambda i: (i, 0)),
            pl.BlockSpec((8, _D), lambda i: (0, 0)),
        ],
        out_shape=[
            jax.ShapeDtypeStruct((_N, _D), jnp.float32),
            jax.ShapeDtypeStruct((8, _D), jnp.float32),
        ],
    )(acc, h, w1, b1.reshape(1, _D))


def _mmB_body(t_ref, st_ref, p_ref, w2_ref, u_ref, st2_ref):
    i = pl.program_id(0)
    mu = st_ref[0:1, :] * (1.0 / _N)
    var = st_ref[1:2, :] * (1.0 / _N) - mu * mu
    inv = lax.rsqrt(var + _EPS)
    h1 = jnp.maximum(p_ref[0:1, :] * (t_ref[...] - mu) * inv + p_ref[1:2, :], 0.0)
    u = jnp.dot(h1, w2_ref[...], preferred_element_type=jnp.float32)
    u = u + p_ref[2:3, :]
    u_ref[...] = u
    s1 = jnp.sum(u, axis=0, keepdims=True)
    s2 = jnp.sum(u * u, axis=0, keepdims=True)
    st = jnp.concatenate([s1, s2, jnp.zeros((6, _D), jnp.float32)], axis=0)

    @pl.when(i == 0)
    def _():
        st2_ref[...] = st

    @pl.when(i != 0)
    def _():
        st2_ref[...] = st2_ref[...] + st


def _mmB(t, st, bng, bnb, w2, b2):
    p = jnp.stack([bng, bnb, b2], axis=0)
    return pl.pallas_call(
        _mmB_body,
        grid=(_N // _BR,),
        in_specs=[
            pl.BlockSpec((_BR, _D), lambda i: (i, 0)),
            pl.BlockSpec((8, _D), lambda i: (0, 0)),
            pl.BlockSpec((3, _D), lambda i: (0, 0)),
            pl.BlockSpec((_D, _D), lambda i: (0, 0)),
        ],
        out_specs=[
            pl.BlockSpec((_BR, _D), lambda i: (i, 0)),
            pl.BlockSpec((8, _D), lambda i: (0, 0)),
        ],
        out_shape=[
            jax.ShapeDtypeStruct((_N, _D), jnp.float32),
            jax.ShapeDtypeStruct((8, _D), jnp.float32),
        ],
    )(t, st, p, w2)


def _bnC_body(u_ref, st_ref, p_ref, h_ref):
    mu = st_ref[0:1, :] * (1.0 / _N)
    var = st_ref[1:2, :] * (1.0 / _N) - mu * mu
    inv = lax.rsqrt(var + _EPS)
    h_ref[...] = jnp.maximum(
        p_ref[0:1, :] * (u_ref[...] - mu) * inv + p_ref[1:2, :], 0.0)


def _bnC(u, st, g, b):
    p = jnp.stack([g, b], axis=0)
    return pl.pallas_call(
        _bnC_body,
        grid=(_N // _BR,),
        in_specs=[
            pl.BlockSpec((_BR, _D), lambda i: (i, 0)),
            pl.BlockSpec((8, _D), lambda i: (0, 0)),
            pl.BlockSpec((2, _D), lambda i: (0, 0)),
        ],
        out_specs=pl.BlockSpec((_BR, _D), lambda i: (i, 0)),
        out_shape=jax.ShapeDtypeStruct((_N, _D), jnp.float32),
    )(u, st, p)


def _bnD_body(u_ref, st_ref, p_ref, s_ref):
    i = pl.program_id(0)
    mu = st_ref[0:1, :] * (1.0 / _N)
    var = st_ref[1:2, :] * (1.0 / _N) - mu * mu
    inv = lax.rsqrt(var + _EPS)
    h = jnp.maximum(p_ref[0:1, :] * (u_ref[...] - mu) * inv + p_ref[1:2, :], 0.0)
    s = jnp.concatenate(
        [jnp.sum(h, axis=0, keepdims=True), jnp.zeros((7, _D), jnp.float32)],
        axis=0)

    @pl.when(i == 0)
    def _():
        s_ref[...] = s

    @pl.when(i != 0)
    def _():
        s_ref[...] = s_ref[...] + s


def _bnD(u, st, g, b):
    p = jnp.stack([g, b], axis=0)
    return pl.pallas_call(
        _bnD_body,
        grid=(_N // _BR,),
        in_specs=[
            pl.BlockSpec((_BR, _D), lambda i: (i, 0)),
            pl.BlockSpec((8, _D), lambda i: (0, 0)),
            pl.BlockSpec((2, _D), lambda i: (0, 0)),
        ],
        out_specs=pl.BlockSpec((8, _D), lambda i: (0, 0)),
        out_shape=jax.ShapeDtypeStruct((8, _D), jnp.float32),
    )(u, st, p)


def _pred_body(s0_ref, s1_ref, s2_ref, pw_ref, pb_ref, o_ref):
    o = jnp.dot(s0_ref[2:3, :], pw_ref[0], preferred_element_type=jnp.float32)
    o = o + jnp.dot(s1_ref[2:3, :], pw_ref[1],
                    preferred_element_type=jnp.float32)
    o = o + jnp.dot(s2_ref[0:1, :], pw_ref[2],
                    preferred_element_type=jnp.float32)
    o_ref[...] = o + pb_ref[0:1, :] + pb_ref[1:2, :] + pb_ref[2:3, :]


def _pred(stA0, stA1, stD, pws, pbs):
    return pl.pallas_call(
        _pred_body,
        out_shape=jax.ShapeDtypeStruct((1, _D), jnp.float32),
    )(stA0, stA1, stD, pws, pbs)


def kernel(x, edge_index,
           l0_w1, l0_b1, l0_bng, l0_bnb, l0_w2, l0_b2, l0_g, l0_b,
           l1_w1, l1_b1, l1_bng, l1_bnb, l1_w2, l1_b2, l1_g, l1_b,
           pred_w0, pred_b0, pred_w1, pred_b1, pred_w2, pred_b2):
    nw, epw = 32, _E // 32
    pad = _NITER * _K - epw
    src = jnp.pad(edge_index[0].reshape(nw, epw), ((0, 0), (0, pad)))
    dst = jnp.pad(edge_index[1].reshape(nw, epw), ((0, 0), (0, pad)),
                  constant_values=_N)
    src = src.reshape(nw, _NITER, _K)
    dst = dst.reshape(nw, _NITER, _K)
    seg_sum = _make_seg_sum()

    c = pred_b0.shape[0]
    pws = jnp.zeros((3, _D, _D), jnp.float32)
    pws = pws.at[0, :, :c].set(pred_w0)
    pws = pws.at[1, :, :c].set(pred_w1)
    pws = pws.at[2, :, :c].set(pred_w2)
    pbs = jnp.zeros((3, _D), jnp.float32)
    pbs = pbs.at[0, :c].set(pred_b0)
    pbs = pbs.at[1, :c].set(pred_b1)
    pbs = pbs.at[2, :c].set(pred_b2)

    acc = seg_sum(x, src, dst).reshape(2, _N, _D)
    t0, stA0 = _mmA(acc, x, l0_w1, l0_b1)
    u0, stB0 = _mmB(t0, stA0, l0_bng, l0_bnb, l0_w2, l0_b2)
    h0 = _bnC(u0, stB0, l0_g, l0_b)

    acc = seg_sum(h0, src, dst).reshape(2, _N, _D)
    t1, stA1 = _mmA(acc, h0, l1_w1, l1_b1)
    u1, stB1 = _mmB(t1, stA1, l1_bng, l1_bnb, l1_w2, l1_b2)
    stD = _bnD(u1, stB1, l1_g, l1_b)

    score = _pred(stA0, stA1, stD, pws, pbs)
    return score[:, :c]


# speedup vs baseline: 1.0004x; 1.0004x over previous
import functools

import jax
import jax.numpy as jnp
from jax import lax
from jax.experimental import pallas as pl
from jax.experimental.pallas import tpu as pltpu
from jax.experimental.pallas import tpu_sc as plsc

_N = 10000
_E = 320000
_D = 128
_EPS = 1e-5
_BR = 1000


_K = 128
_NITER = 80


def _make_seg_sum():
    nc, ns = 2, 16
    k = _K
    niter = _NITER
    ch = 80
    ncht = _N // ch
    tpt = -(-ncht // ns)
    nacc = _N + 1024
    mesh = plsc.VectorSubcoreMesh(
        core_axis_name="c", subcore_axis_name="s",
        num_cores=nc, num_subcores=ns)

    @functools.partial(
        pl.kernel,
        mesh=mesh,
        out_type=jax.ShapeDtypeStruct((2 * _N, _D), jnp.float32),
        scratch_types=[
            pltpu.MemorySpace.VMEM_SHARED((nacc, _D), jnp.float32),
            pltpu.MemorySpace.VMEM((niter, k), jnp.int32),
            pltpu.MemorySpace.VMEM((niter, k), jnp.int32),
            pltpu.MemorySpace.VMEM((k, _D), jnp.float32),
            pltpu.SemaphoreType.DMA,
            pltpu.SemaphoreType.DMA,
        ],
    )
    def seg_sum(h_hbm, src_hbm, dst_hbm, out_hbm, acc, sidx, didx, rows,
                gsem, isem):
        cid = lax.axis_index("c")
        sid = lax.axis_index("s")
        wid = sid * nc + cid

        cps = pltpu.async_copy(src_hbm.at[wid], sidx, isem)
        cpd = pltpu.async_copy(dst_hbm.at[wid], didx, isem)

        def init_body(t, _):
            cix = sid + t * ns

            @pl.when(cix < ncht)
            def _():
                r0 = cix * ch
                pltpu.sync_copy(h_hbm.at[pl.ds(r0, ch)],
                                rows.at[pl.ds(0, ch)])
                pltpu.sync_copy(rows.at[pl.ds(0, ch)],
                                acc.at[pl.ds(r0, ch)])
            return 0

        lax.fori_loop(0, tpt, init_body, 0)
        cps.wait()
        cpd.wait()
        plsc.subcore_barrier()

        def edge_body(j, _):
            pltpu.async_copy(h_hbm.at[sidx.at[j]], rows, gsem).wait()
            pltpu.sync_copy(rows, acc.at[didx.at[j]], add=True)
            return 0

        lax.fori_loop(0, niter, edge_body, 0)
        plsc.subcore_barrier()

        def wb_body(t, _):
            cix = sid + t * ns

            @pl.when(cix < ncht)
            def _():
                r0 = cix * ch
                pltpu.sync_copy(acc.at[pl.ds(r0, ch)],
                                rows.at[pl.ds(0, ch)])
                pltpu.sync_copy(rows.at[pl.ds(0, ch)],
                                out_hbm.at[pl.ds(cid * _N + r0, ch)])
            return 0

        lax.fori_loop(0, tpt, wb_body, 0)

    return seg_sum


def _mmA_body(acc_ref, h_ref, w_ref, b_ref, t_ref, st_ref):
    i = pl.program_id(0)
    pooled = acc_ref[0] + acc_ref[1] - h_ref[...]
    t = jnp.dot(pooled, w_ref[...], preferred_element_type=jnp.float32)
    t = t + b_ref[0:1, :]
    t_ref[...] = t
    s1 = jnp.sum(t, axis=0, keepdims=True)
    s2 = jnp.sum(t * t, axis=0, keepdims=True)
    s3 = jnp.sum(h_ref[...], axis=0, keepdims=True)
    st = jnp.concatenate(
        [s1, s2, s3, jnp.zeros((5, _D), jnp.float32)], axis=0)

    @pl.when(i == 0)
    def _():
        st_ref[...] = st

    @pl.when(i != 0)
    def _():
        st_ref[...] = st_ref[...] + st


def _mmA(acc, h, w1, b1):
    return pl.pallas_call(
        _mmA_body,
        grid=(_N // _BR,),
        in_specs=[
            pl.BlockSpec((2, _BR, _D), lambda i: (0, i, 0)),
            pl.BlockSpec((_BR, _D), lambda i: (i, 0)),
            pl.BlockSpec((_D, _D), lambda i: (0, 0)),
            pl.BlockSpec((1, _D), lambda i: (0, 0)),
        ],
        out_specs=[
            pl.BlockSpec((_BR, _D), lambda i: (i, 0)),
            pl.BlockSpec((8, _D), lambda i: (0, 0)),
        ],
        out_shape=[
            jax.ShapeDtypeStruct((_N, _D), jnp.float32),
            jax.ShapeDtypeStruct((8, _D), jnp.float32),
        ],
    )(acc, h, w1, b1.reshape(1, _D))


def _mmB_body(t_ref, st_ref, p_ref, w2_ref, u_ref, st2_ref):
    i = pl.program_id(0)
    mu = st_ref[0:1, :] * (1.0 / _N)
    var = st_ref[1:2, :] * (1.0 / _N) - mu * mu
    inv = lax.rsqrt(var + _EPS)
    h1 = jnp.maximum(p_ref[0:1, :] * (t_ref[...] - mu) * inv + p_ref[1:2, :], 0.0)
    u = jnp.dot(h1, w2_ref[...], preferred_element_type=jnp.float32)
    u = u + p_ref[2:3, :]
    u_ref[...] = u
    s1 = jnp.sum(u, axis=0, keepdims=True)
    s2 = jnp.sum(u * u, axis=0, keepdims=True)
    st = jnp.concatenate([s1, s2, jnp.zeros((6, _D), jnp.float32)], axis=0)

    @pl.when(i == 0)
    def _():
        st2_ref[...] = st

    @pl.when(i != 0)
    def _():
        st2_ref[...] = st2_ref[...] + st


def _mmB(t, st, bng, bnb, w2, b2):
    p = jnp.stack([bng, bnb, b2], axis=0)
    return pl.pallas_call(
        _mmB_body,
        grid=(_N // _BR,),
        in_specs=[
            pl.BlockSpec((_BR, _D), lambda i: (i, 0)),
            pl.BlockSpec((8, _D), lambda i: (0, 0)),
            pl.BlockSpec((3, _D), lambda i: (0, 0)),
            pl.BlockSpec((_D, _D), lambda i: (0, 0)),
        ],
        out_specs=[
            pl.BlockSpec((_BR, _D), lambda i: (i, 0)),
            pl.BlockSpec((8, _D), lambda i: (0, 0)),
        ],
        out_shape=[
            jax.ShapeDtypeStruct((_N, _D), jnp.float32),
            jax.ShapeDtypeStruct((8, _D), jnp.float32),
        ],
    )(t, st, p, w2)


def _bnC_body(u_ref, st_ref, p_ref, h_ref):
    mu = st_ref[0:1, :] * (1.0 / _N)
    var = st_ref[1:2, :] * (1.0 / _N) - mu * mu
    inv = lax.rsqrt(var + _EPS)
    h_ref[...] = jnp.maximum(
        p_ref[0:1, :] * (u_ref[...] - mu) * inv + p_ref[1:2, :], 0.0)


def _bnC(u, st, g, b):
    p = jnp.stack([g, b], axis=0)
    return pl.pallas_call(
        _bnC_body,
        grid=(_N // _BR,),
        in_specs=[
            pl.BlockSpec((_BR, _D), lambda i: (i, 0)),
            pl.BlockSpec((8, _D), lambda i: (0, 0)),
            pl.BlockSpec((2, _D), lambda i: (0, 0)),
        ],
        out_specs=pl.BlockSpec((_BR, _D), lambda i: (i, 0)),
        out_shape=jax.ShapeDtypeStruct((_N, _D), jnp.float32),
    )(u, st, p)


def _bnD_body(u_ref, st_ref, p_ref, s_ref):
    i = pl.program_id(0)
    mu = st_ref[0:1, :] * (1.0 / _N)
    var = st_ref[1:2, :] * (1.0 / _N) - mu * mu
    inv = lax.rsqrt(var + _EPS)
    h = jnp.maximum(p_ref[0:1, :] * (u_ref[...] - mu) * inv + p_ref[1:2, :], 0.0)
    s = jnp.concatenate(
        [jnp.sum(h, axis=0, keepdims=True), jnp.zeros((7, _D), jnp.float32)],
        axis=0)

    @pl.when(i == 0)
    def _():
        s_ref[...] = s

    @pl.when(i != 0)
    def _():
        s_ref[...] = s_ref[...] + s


def _bnD(u, st, g, b):
    p = jnp.stack([g, b], axis=0)
    return pl.pallas_call(
        _bnD_body,
        grid=(_N // _BR,),
        in_specs=[
            pl.BlockSpec((_BR, _D), lambda i: (i, 0)),
            pl.BlockSpec((8, _D), lambda i: (0, 0)),
            pl.BlockSpec((2, _D), lambda i: (0, 0)),
        ],
        out_specs=pl.BlockSpec((8, _D), lambda i: (0, 0)),
        out_shape=jax.ShapeDtypeStruct((8, _D), jnp.float32),
    )(u, st, p)


def _pred_body(s0_ref, s1_ref, s2_ref, pw_ref, pb_ref, o_ref):
    o = jnp.dot(s0_ref[2:3, :], pw_ref[0], preferred_element_type=jnp.float32)
    o = o + jnp.dot(s1_ref[2:3, :], pw_ref[1],
                    preferred_element_type=jnp.float32)
    o = o + jnp.dot(s2_ref[0:1, :], pw_ref[2],
                    preferred_element_type=jnp.float32)
    o_ref[...] = o + pb_ref[0:1, :] + pb_ref[1:2, :] + pb_ref[2:3, :]


def _pred(stA0, stA1, stD, pws, pbs):
    return pl.pallas_call(
        _pred_body,
        out_shape=jax.ShapeDtypeStruct((1, _D), jnp.float32),
    )(stA0, stA1, stD, pws, pbs)


def kernel(x, edge_index,
           l0_w1, l0_b1, l0_bng, l0_bnb, l0_w2, l0_b2, l0_g, l0_b,
           l1_w1, l1_b1, l1_bng, l1_bnb, l1_w2, l1_b2, l1_g, l1_b,
           pred_w0, pred_b0, pred_w1, pred_b1, pred_w2, pred_b2):
    nw, epw = 32, _E // 32
    pad = _NITER * _K - epw
    src = jnp.pad(edge_index[0].reshape(nw, epw), ((0, 0), (0, pad)))
    waste = _N + (jnp.arange(nw * pad, dtype=jnp.int32) % 1024)
    dst = jnp.concatenate(
        [edge_index[1].reshape(nw, epw), waste.reshape(nw, pad)], axis=1)
    src = src.reshape(nw, _NITER, _K)
    dst = dst.reshape(nw, _NITER, _K)
    seg_sum = _make_seg_sum()

    c = pred_b0.shape[0]
    pws = jnp.zeros((3, _D, _D), jnp.float32)
    pws = pws.at[0, :, :c].set(pred_w0)
    pws = pws.at[1, :, :c].set(pred_w1)
    pws = pws.at[2, :, :c].set(pred_w2)
    pbs = jnp.zeros((3, _D), jnp.float32)
    pbs = pbs.at[0, :c].set(pred_b0)
    pbs = pbs.at[1, :c].set(pred_b1)
    pbs = pbs.at[2, :c].set(pred_b2)

    acc = seg_sum(x, src, dst).reshape(2, _N, _D)
    t0, stA0 = _mmA(acc, x, l0_w1, l0_b1)
    u0, stB0 = _mmB(t0, stA0, l0_bng, l0_bnb, l0_w2, l0_b2)
    h0 = _bnC(u0, stB0, l0_g, l0_b)

    acc = seg_sum(h0, src, dst).reshape(2, _N, _D)
    t1, stA1 = _mmA(acc, h0, l1_w1, l1_b1)
    u1, stB1 = _mmB(t1, stA1, l1_bng, l1_bnb, l1_w2, l1_b2)
    stD = _bnD(u1, stB1, l1_g, l1_b)

    score = _pred(stA0, stA1, stD, pws, pbs)
    return score[:, :c]


# speedup vs baseline: 2.9914x; 2.9904x over previous
import functools

import jax
import jax.numpy as jnp
from jax import lax
from jax.experimental import pallas as pl
from jax.experimental.pallas import tpu as pltpu
from jax.experimental.pallas import tpu_sc as plsc

_N = 10000
_E = 320000
_D = 128
_EPS = 1e-5
_BR = 1000


_K = 80
_NITER = 125


def _make_seg_sum():
    nc, ns = 2, 16
    k = _K
    niter = _NITER
    ch = 80
    ncht = _N // ch
    tpt = -(-ncht // ns)
    mesh = plsc.VectorSubcoreMesh(
        core_axis_name="c", subcore_axis_name="s",
        num_cores=nc, num_subcores=ns)

    @functools.partial(
        pl.kernel,
        mesh=mesh,
        out_type=jax.ShapeDtypeStruct((2 * _N, _D), jnp.float32),
        scratch_types=[
            pltpu.MemorySpace.VMEM_SHARED((_N, _D), jnp.float32),
            pltpu.MemorySpace.VMEM((niter * k,), jnp.int32),
            pltpu.MemorySpace.VMEM((niter, k), jnp.int32),
            [pltpu.MemorySpace.VMEM((k, _D), jnp.float32)] * 2,
            [pltpu.SemaphoreType.DMA] * 2,
            pltpu.SemaphoreType.DMA,
        ],
    )
    def seg_sum(h_hbm, src_hbm, dst_hbm, out_hbm, acc, sidx, didx, rows,
                gsem, isem):
        cid = lax.axis_index("c")
        sid = lax.axis_index("s")
        wid = sid * nc + cid

        cps = pltpu.async_copy(src_hbm.at[wid], sidx, isem)
        cpd = pltpu.async_copy(dst_hbm.at[wid], didx, isem)

        def init_body(t, _):
            cix = sid + t * ns

            @pl.when(cix < ncht)
            def _():
                r0 = cix * ch
                pltpu.sync_copy(h_hbm.at[pl.ds(r0, ch)],
                                rows[0].at[pl.ds(0, ch)])
                pltpu.sync_copy(rows[0].at[pl.ds(0, ch)],
                                acc.at[pl.ds(r0, ch)])
            return 0

        lax.fori_loop(0, tpt, init_body, 0)
        cps.wait()
        cpd.wait()
        plsc.subcore_barrier()

        def gather(j, b):
            return pltpu.async_copy(h_hbm.at[sidx.at[pl.ds(j * k, k)]],
                                    rows[b], gsem[b])

        gather(0, 0)
        gather(1, 1)

        def edge_body(g, _):
            j0 = 2 * g
            for b in range(2):
                pltpu.make_async_copy(h_hbm.at[sidx.at[pl.ds(0, k)]],
                                      rows[b], gsem[b]).wait()
                pltpu.sync_copy(rows[b], acc.at[didx.at[j0 + b]], add=True)
                gather(j0 + b + 2, b)
            return 0

        lax.fori_loop(0, (niter - 3) // 2, edge_body, 0)

        t0 = niter - 3
        for j in range(t0, niter):
            b = j % 2
            pltpu.make_async_copy(h_hbm.at[sidx.at[pl.ds(0, k)]],
                                  rows[b], gsem[b]).wait()
            pltpu.sync_copy(rows[b], acc.at[didx.at[j]], add=True)
            if j + 2 < niter:
                gather(j + 2, b)
        plsc.subcore_barrier()

        def wb_body(t, _):
            cix = sid + t * ns

            @pl.when(cix < ncht)
            def _():
                r0 = cix * ch
                pltpu.sync_copy(acc.at[pl.ds(r0, ch)],
                                rows[0].at[pl.ds(0, ch)])
                pltpu.sync_copy(rows[0].at[pl.ds(0, ch)],
                                out_hbm.at[pl.ds(cid * _N + r0, ch)])
            return 0

        lax.fori_loop(0, tpt, wb_body, 0)

    return seg_sum


def _mmA_body(acc_ref, h_ref, w_ref, b_ref, t_ref, st_ref):
    i = pl.program_id(0)
    pooled = acc_ref[0] + acc_ref[1] - h_ref[...]
    t = jnp.dot(pooled, w_ref[...], preferred_element_type=jnp.float32)
    t = t + b_ref[0:1, :]
    t_ref[...] = t
    s1 = jnp.sum(t, axis=0, keepdims=True)
    s2 = jnp.sum(t * t, axis=0, keepdims=True)
    s3 = jnp.sum(h_ref[...], axis=0, keepdims=True)
    st = jnp.concatenate(
        [s1, s2, s3, jnp.zeros((5, _D), jnp.float32)], axis=0)

    @pl.when(i == 0)
    def _():
        st_ref[...] = st

    @pl.when(i != 0)
    def _():
        st_ref[...] = st_ref[...] + st


def _mmA(acc, h, w1, b1):
    return pl.pallas_call(
        _mmA_body,
        grid=(_N // _BR,),
        in_specs=[
            pl.BlockSpec((2, _BR, _D), lambda i: (0, i, 0)),
            pl.BlockSpec((_BR, _D), lambda i: (i, 0)),
            pl.BlockSpec((_D, _D), lambda i: (0, 0)),
            pl.BlockSpec((1, _D), lambda i: (0, 0)),
        ],
        out_specs=[
            pl.BlockSpec((_BR, _D), lambda i: (i, 0)),
            pl.BlockSpec((8, _D), lambda i: (0, 0)),
        ],
        out_shape=[
            jax.ShapeDtypeStruct((_N, _D), jnp.float32),
            jax.ShapeDtypeStruct((8, _D), jnp.float32),
        ],
    )(acc, h, w1, b1.reshape(1, _D))


def _mmB_body(t_ref, st_ref, p_ref, w2_ref, u_ref, st2_ref):
    i = pl.program_id(0)
    mu = st_ref[0:1, :] * (1.0 / _N)
    var = st_ref[1:2, :] * (1.0 / _N) - mu * mu
    inv = lax.rsqrt(var + _EPS)
    h1 = jnp.maximum(p_ref[0:1, :] * (t_ref[...] - mu) * inv + p_ref[1:2, :], 0.0)
    u = jnp.dot(h1, w2_ref[...], preferred_element_type=jnp.float32)
    u = u + p_ref[2:3, :]
    u_ref[...] = u
    s1 = jnp.sum(u, axis=0, keepdims=True)
    s2 = jnp.sum(u * u, axis=0, keepdims=True)
    st = jnp.concatenate([s1, s2, jnp.zeros((6, _D), jnp.float32)], axis=0)

    @pl.when(i == 0)
    def _():
        st2_ref[...] = st

    @pl.when(i != 0)
    def _():
        st2_ref[...] = st2_ref[...] + st


def _mmB(t, st, bng, bnb, w2, b2):
    p = jnp.stack([bng, bnb, b2], axis=0)
    return pl.pallas_call(
        _mmB_body,
        grid=(_N // _BR,),
        in_specs=[
            pl.BlockSpec((_BR, _D), lambda i: (i, 0)),
            pl.BlockSpec((8, _D), lambda i: (0, 0)),
            pl.BlockSpec((3, _D), lambda i: (0, 0)),
            pl.BlockSpec((_D, _D), lambda i: (0, 0)),
        ],
        out_specs=[
            pl.BlockSpec((_BR, _D), lambda i: (i, 0)),
            pl.BlockSpec((8, _D), lambda i: (0, 0)),
        ],
        out_shape=[
            jax.ShapeDtypeStruct((_N, _D), jnp.float32),
            jax.ShapeDtypeStruct((8, _D), jnp.float32),
        ],
    )(t, st, p, w2)


def _bnC_body(u_ref, st_ref, p_ref, h_ref):
    mu = st_ref[0:1, :] * (1.0 / _N)
    var = st_ref[1:2, :] * (1.0 / _N) - mu * mu
    inv = lax.rsqrt(var + _EPS)
    h_ref[...] = jnp.maximum(
        p_ref[0:1, :] * (u_ref[...] - mu) * inv + p_ref[1:2, :], 0.0)


def _bnC(u, st, g, b):
    p = jnp.stack([g, b], axis=0)
    return pl.pallas_call(
        _bnC_body,
        grid=(_N // _BR,),
        in_specs=[
            pl.BlockSpec((_BR, _D), lambda i: (i, 0)),
            pl.BlockSpec((8, _D), lambda i: (0, 0)),
            pl.BlockSpec((2, _D), lambda i: (0, 0)),
        ],
        out_specs=pl.BlockSpec((_BR, _D), lambda i: (i, 0)),
        out_shape=jax.ShapeDtypeStruct((_N, _D), jnp.float32),
    )(u, st, p)


def _bnD_body(u_ref, st_ref, p_ref, s_ref):
    i = pl.program_id(0)
    mu = st_ref[0:1, :] * (1.0 / _N)
    var = st_ref[1:2, :] * (1.0 / _N) - mu * mu
    inv = lax.rsqrt(var + _EPS)
    h = jnp.maximum(p_ref[0:1, :] * (u_ref[...] - mu) * inv + p_ref[1:2, :], 0.0)
    s = jnp.concatenate(
        [jnp.sum(h, axis=0, keepdims=True), jnp.zeros((7, _D), jnp.float32)],
        axis=0)

    @pl.when(i == 0)
    def _():
        s_ref[...] = s

    @pl.when(i != 0)
    def _():
        s_ref[...] = s_ref[...] + s


def _bnD(u, st, g, b):
    p = jnp.stack([g, b], axis=0)
    return pl.pallas_call(
        _bnD_body,
        grid=(_N // _BR,),
        in_specs=[
            pl.BlockSpec((_BR, _D), lambda i: (i, 0)),
            pl.BlockSpec((8, _D), lambda i: (0, 0)),
            pl.BlockSpec((2, _D), lambda i: (0, 0)),
        ],
        out_specs=pl.BlockSpec((8, _D), lambda i: (0, 0)),
        out_shape=jax.ShapeDtypeStruct((8, _D), jnp.float32),
    )(u, st, p)


def _pred_body(s0_ref, s1_ref, s2_ref, pw_ref, pb_ref, o_ref):
    o = jnp.dot(s0_ref[2:3, :], pw_ref[0], preferred_element_type=jnp.float32)
    o = o + jnp.dot(s1_ref[2:3, :], pw_ref[1],
                    preferred_element_type=jnp.float32)
    o = o + jnp.dot(s2_ref[0:1, :], pw_ref[2],
                    preferred_element_type=jnp.float32)
    o_ref[...] = o + pb_ref[0:1, :] + pb_ref[1:2, :] + pb_ref[2:3, :]


def _pred(stA0, stA1, stD, pws, pbs):
    return pl.pallas_call(
        _pred_body,
        out_shape=jax.ShapeDtypeStruct((1, _D), jnp.float32),
    )(stA0, stA1, stD, pws, pbs)


def kernel(x, edge_index,
           l0_w1, l0_b1, l0_bng, l0_bnb, l0_w2, l0_b2, l0_g, l0_b,
           l1_w1, l1_b1, l1_bng, l1_bnb, l1_w2, l1_b2, l1_g, l1_b,
           pred_w0, pred_b0, pred_w1, pred_b1, pred_w2, pred_b2):
    nw, epw = 32, _E // 32
    src = edge_index[0].reshape(nw, epw)
    dst = edge_index[1].reshape(nw, _NITER, _K)
    seg_sum = _make_seg_sum()

    c = pred_b0.shape[0]
    pws = jnp.zeros((3, _D, _D), jnp.float32)
    pws = pws.at[0, :, :c].set(pred_w0)
    pws = pws.at[1, :, :c].set(pred_w1)
    pws = pws.at[2, :, :c].set(pred_w2)
    pbs = jnp.zeros((3, _D), jnp.float32)
    pbs = pbs.at[0, :c].set(pred_b0)
    pbs = pbs.at[1, :c].set(pred_b1)
    pbs = pbs.at[2, :c].set(pred_b2)

    acc = seg_sum(x, src, dst).reshape(2, _N, _D)
    t0, stA0 = _mmA(acc, x, l0_w1, l0_b1)
    u0, stB0 = _mmB(t0, stA0, l0_bng, l0_bnb, l0_w2, l0_b2)
    h0 = _bnC(u0, stB0, l0_g, l0_b)

    acc = seg_sum(h0, src, dst).reshape(2, _N, _D)
    t1, stA1 = _mmA(acc, h0, l1_w1, l1_b1)
    u1, stB1 = _mmB(t1, stA1, l1_bng, l1_bnb, l1_w2, l1_b2)
    stD = _bnD(u1, stB1, l1_g, l1_b)

    score = _pred(stA0, stA1, stD, pws, pbs)
    return score[:, :c]
